# baseline (speedup 1.0000x reference)
import functools
import jax
import jax.numpy as jnp
from jax import lax
from jax.experimental import pallas as pl
from jax.experimental.pallas import tpu as pltpu
from jax.experimental.pallas import tpu_sc as plsc

NW = 32
L = 16
N = 4 * 512 * 512
PER_W = N // NW
CH = 16384
NCHUNK = PER_W // CH
NCOPY = 4
NBIN = 4096
NEG_OFF = -103 * 64
POS_OFF = NEG_OFF + 2048

_mesh = plsc.VectorSubcoreMesh(core_axis_name="c", subcore_axis_name="s")


def _sc_body(pred_hbm, t_hbm, m_hbm, cnt_out, sum_out, pred_c, t_c, m_c,
             c0, c1, c2, c3, s0, s1, s2, s3):
    wid = lax.axis_index("s") * 2 + lax.axis_index("c")
    base = wid * PER_W
    cs = ((c0, s0), (c1, s1), (c2, s2), (c3, s3))

    zeros = jnp.zeros((L,), jnp.float32)

    def zero_hist(i, _):
        o = i * L
        for cv, sv in cs:
            cv[pl.ds(o, L)] = zeros
            sv[pl.ds(o, L)] = zeros
        return 0

    lax.fori_loop(0, NBIN // L, zero_hist, 0, unroll=4)

    ones = jnp.ones((L,), jnp.float32)

    def chunk(ch, _):
        off = base + ch * CH
        pltpu.sync_copy(pred_hbm.at[pl.ds(off, CH)], pred_c)
        pltpu.sync_copy(t_hbm.at[pl.ds(off, CH)], t_c)
        pltpu.sync_copy(m_hbm.at[pl.ds(off, CH)], m_c)

        def vec(i, _):
            for j, (cv, sv) in enumerate(cs):
                o = (i * NCOPY + j) * L
                p = pred_c[pl.ds(o, L)]
                t = t_c[pl.ds(o, L)]
                m = m_c[pl.ds(o, L)]
                tpos = t > 0
                q = jnp.maximum(jnp.where(tpos, p, 1.0 - p), 1e-7)
                key = (plsc.bitcast(q, jnp.int32) >> 17) + jnp.where(
                    tpos, POS_OFF, NEG_OFF)
                msk = m > 0
                plsc.addupdate_scatter(cv, [key], ones, mask=msk)
                plsc.addupdate_scatter(sv, [key], q, mask=msk)
            return 0

        lax.fori_loop(0, CH // L // NCOPY, vec, 0)
        return 0

    lax.fori_loop(0, NCHUNK, chunk, 0)

    def merge(i, _):
        o = i * L
        c0[pl.ds(o, L)] = (c0[pl.ds(o, L)] + c1[pl.ds(o, L)]) + (
            c2[pl.ds(o, L)] + c3[pl.ds(o, L)])
        s0[pl.ds(o, L)] = (s0[pl.ds(o, L)] + s1[pl.ds(o, L)]) + (
            s2[pl.ds(o, L)] + s3[pl.ds(o, L)])
        return 0

    lax.fori_loop(0, NBIN // L, merge, 0, unroll=4)
    pltpu.sync_copy(c0, cnt_out.at[wid])
    pltpu.sync_copy(s0, sum_out.at[wid])


_sc_hist = functools.partial(
    pl.kernel, mesh=_mesh,
    out_type=(
        jax.ShapeDtypeStruct((NW, NBIN), jnp.float32),
        jax.ShapeDtypeStruct((NW, NBIN), jnp.float32),
    ),
    scratch_types=[
        pltpu.VMEM((CH,), jnp.float32),
        pltpu.VMEM((CH,), jnp.int32),
        pltpu.VMEM((CH,), jnp.int32),
    ] + [pltpu.VMEM((NBIN,), jnp.float32)] * 8,
    compiler_params=pltpu.CompilerParams(needs_layout_passes=False),
)(_sc_body)


def _tc_select_body(cnt_ref, sum_ref, out_ref):
    A = jnp.sum(cnt_ref[...], axis=0)
    Q = jnp.sum(sum_ref[...], axis=0)
    C2 = A[0:16, :]
    Q2 = Q[0:16, :]
    posC = A[16:32, :]
    posQ = Q[16:32, :]
    npos = jnp.sum(posC)
    qm_pos = jnp.where(posC > 0.0, posQ / jnp.maximum(posC, 1.0), 0.5)
    lpos = jnp.sum(posC * -jnp.log(qm_pos))
    qm_neg = jnp.where(C2 > 0.0, Q2 / jnp.maximum(C2, 1.0), 0.5)
    S2 = C2 * -jnp.log(qm_neg)
    hp = jax.lax.Precision.HIGHEST
    M1 = (lax.broadcasted_iota(jnp.int32, (128, 128), 0)
          <= lax.broadcasted_iota(jnp.int32, (128, 128), 1)).astype(jnp.float32)
    PrefC = jnp.dot(C2, M1, preferred_element_type=jnp.float32, precision=hp)
    PrefS = jnp.dot(S2, M1, preferred_element_type=jnp.float32, precision=hp)
    Arr = (lax.broadcasted_iota(jnp.int32, (16, 16), 1)
           < lax.broadcasted_iota(jnp.int32, (16, 16), 0)).astype(jnp.float32)
    RowC = jnp.dot(Arr, PrefC[:, 127:128], preferred_element_type=jnp.float32,
                   precision=hp)
    RowS = jnp.dot(Arr, PrefS[:, 127:128], preferred_element_type=jnp.float32,
                   precision=hp)
    C_geq = RowC + PrefC
    C_above = C_geq - C2
    S_above = RowS + PrefS - S2
    nneg = jnp.sum(C2)
    k = jnp.where(npos > 0.0, jnp.minimum(nneg, 3.0 * npos), 100.0)
    k_eff = jnp.minimum(k, nneg)
    sel = jnp.logical_and(C_above < k_eff, C_geq >= k_eff)
    self32 = jnp.where(sel, 1.0, 0.0) * jnp.where(k_eff > 0.0, 1.0, 0.0)
    cnt_sel = jnp.sum(self32 * C2)
    sum_sel = jnp.sum(self32 * S2)
    C_a = jnp.sum(self32 * C_above)
    S_a = jnp.sum(self32 * S_above)
    mean_sel = sum_sel / jnp.maximum(cnt_sel, 1.0)
    loss_neg = jnp.where(k_eff > 0.0, S_a + (k_eff - C_a) * mean_sel, 0.0)
    loss_neg = loss_neg + jnp.where(k > nneg, (k - nneg) * -1e30, 0.0)
    out_ref[0, 0] = (lpos + loss_neg) / (npos + k)


def _tc_select(cnt2, sum2):
    out = pl.pallas_call(
        _tc_select_body,
        out_specs=pl.BlockSpec(memory_space=pltpu.SMEM),
        out_shape=jax.ShapeDtypeStruct((1, 1), jnp.float32),
    )(cnt2.reshape(NW, 32, 128), sum2.reshape(NW, 32, 128))
    return out[0, 0]


def kernel(pred, target, train_mask):
    cnt2, sum2 = _sc_hist(
        pred.reshape(-1), target.reshape(-1), train_mask.reshape(-1))
    return _tc_select(cnt2, sum2)


# speedup vs baseline: 1.2266x; 1.2266x over previous
import functools
import jax
import jax.numpy as jnp
from jax import lax
from jax.experimental import pallas as pl
from jax.experimental.pallas import tpu as pltpu
from jax.experimental.pallas import tpu_sc as plsc

NW = 32
L = 16
N = 4 * 512 * 512
PER_W = N // NW
CH = 16384
NCHUNK = PER_W // CH
NCOPY = 4
NBIN = 4096
NEG_OFF = -103 * 64
POS_OFF = NEG_OFF + 2048

_mesh = plsc.VectorSubcoreMesh(core_axis_name="c", subcore_axis_name="s")


def _sc_body(pred_hbm, t_hbm, m_hbm, cnt_out, sum_out, pred_c, t_c, m_c,
             c0, c1, c2, c3, s0, s1, s2, s3):
    wid = lax.axis_index("s") * 2 + lax.axis_index("c")
    base = wid * PER_W
    cs = ((c0, s0), (c1, s1), (c2, s2), (c3, s3))

    zeros = jnp.zeros((L,), jnp.float32)

    def zero_hist(i, _):
        o = i * L
        for cv, sv in cs:
            cv[pl.ds(o, L)] = zeros
            sv[pl.ds(o, L)] = zeros
        return 0

    lax.fori_loop(0, NBIN // L, zero_hist, 0, unroll=4)

    ones = jnp.ones((L,), jnp.float32)

    def chunk(ch, _):
        off = base + ch * CH
        pltpu.sync_copy(pred_hbm.at[pl.ds(off, CH)], pred_c)
        pltpu.sync_copy(t_hbm.at[pl.ds(off, CH)], t_c)
        pltpu.sync_copy(m_hbm.at[pl.ds(off, CH)], m_c)

        def vec(i, _):
            loads = []
            for j in range(NCOPY):
                o = (i * NCOPY + j) * L
                loads.append((pred_c[pl.ds(o, L)], t_c[pl.ds(o, L)],
                              m_c[pl.ds(o, L)]))
            work = []
            for p, t, m in loads:
                tpos = t > 0
                q = jnp.maximum(jnp.where(tpos, p, 1.0 - p), 1e-7)
                key = (plsc.bitcast(q, jnp.int32) >> 17) + jnp.where(
                    tpos, POS_OFF, NEG_OFF)
                work.append((key, q, m > 0))
            for (cv, sv), (key, q, msk) in zip(cs, work):
                plsc.addupdate_scatter(cv, [key], ones, mask=msk)
                plsc.addupdate_scatter(sv, [key], q, mask=msk)
            return 0

        lax.fori_loop(0, CH // L // NCOPY, vec, 0)
        return 0

    lax.fori_loop(0, NCHUNK, chunk, 0)

    def merge(i, _):
        o = i * L
        c0[pl.ds(o, L)] = (c0[pl.ds(o, L)] + c1[pl.ds(o, L)]) + (
            c2[pl.ds(o, L)] + c3[pl.ds(o, L)])
        s0[pl.ds(o, L)] = (s0[pl.ds(o, L)] + s1[pl.ds(o, L)]) + (
            s2[pl.ds(o, L)] + s3[pl.ds(o, L)])
        return 0

    lax.fori_loop(0, NBIN // L, merge, 0, unroll=4)
    pltpu.sync_copy(c0, cnt_out.at[wid])
    pltpu.sync_copy(s0, sum_out.at[wid])


_sc_hist = functools.partial(
    pl.kernel, mesh=_mesh,
    out_type=(
        jax.ShapeDtypeStruct((NW, NBIN), jnp.float32),
        jax.ShapeDtypeStruct((NW, NBIN), jnp.float32),
    ),
    scratch_types=[
        pltpu.VMEM((CH,), jnp.float32),
        pltpu.VMEM((CH,), jnp.int32),
        pltpu.VMEM((CH,), jnp.int32),
    ] + [pltpu.VMEM((NBIN,), jnp.float32)] * 8,
    compiler_params=pltpu.CompilerParams(needs_layout_passes=False),
)(_sc_body)


def _tc_select_body(cnt_ref, sum_ref, out_ref):
    A = jnp.sum(cnt_ref[...], axis=0)
    Q = jnp.sum(sum_ref[...], axis=0)
    C2 = A[0:16, :]
    Q2 = Q[0:16, :]
    posC = A[16:32, :]
    posQ = Q[16:32, :]
    npos = jnp.sum(posC)
    qm_pos = jnp.where(posC > 0.0, posQ / jnp.maximum(posC, 1.0), 0.5)
    lpos = jnp.sum(posC * -jnp.log(qm_pos))
    qm_neg = jnp.where(C2 > 0.0, Q2 / jnp.maximum(C2, 1.0), 0.5)
    S2 = C2 * -jnp.log(qm_neg)
    hp = jax.lax.Precision.HIGHEST
    M1 = (lax.broadcasted_iota(jnp.int32, (128, 128), 0)
          <= lax.broadcasted_iota(jnp.int32, (128, 128), 1)).astype(jnp.float32)
    PrefC = jnp.dot(C2, M1, preferred_element_type=jnp.float32, precision=hp)
    PrefS = jnp.dot(S2, M1, preferred_element_type=jnp.float32, precision=hp)
    Arr = (lax.broadcasted_iota(jnp.int32, (16, 16), 1)
           < lax.broadcasted_iota(jnp.int32, (16, 16), 0)).astype(jnp.float32)
    RowC = jnp.dot(Arr, PrefC[:, 127:128], preferred_element_type=jnp.float32,
                   precision=hp)
    RowS = jnp.dot(Arr, PrefS[:, 127:128], preferred_element_type=jnp.float32,
                   precision=hp)
    C_geq = RowC + PrefC
    C_above = C_geq - C2
    S_above = RowS + PrefS - S2
    nneg = jnp.sum(C2)
    k = jnp.where(npos > 0.0, jnp.minimum(nneg, 3.0 * npos), 100.0)
    k_eff = jnp.minimum(k, nneg)
    sel = jnp.logical_and(C_above < k_eff, C_geq >= k_eff)
    self32 = jnp.where(sel, 1.0, 0.0) * jnp.where(k_eff > 0.0, 1.0, 0.0)
    cnt_sel = jnp.sum(self32 * C2)
    sum_sel = jnp.sum(self32 * S2)
    C_a = jnp.sum(self32 * C_above)
    S_a = jnp.sum(self32 * S_above)
    mean_sel = sum_sel / jnp.maximum(cnt_sel, 1.0)
    loss_neg = jnp.where(k_eff > 0.0, S_a + (k_eff - C_a) * mean_sel, 0.0)
    loss_neg = loss_neg + jnp.where(k > nneg, (k - nneg) * -1e30, 0.0)
    out_ref[0, 0] = (lpos + loss_neg) / (npos + k)


def _tc_select(cnt2, sum2):
    out = pl.pallas_call(
        _tc_select_body,
        out_specs=pl.BlockSpec(memory_space=pltpu.SMEM),
        out_shape=jax.ShapeDtypeStruct((1, 1), jnp.float32),
    )(cnt2.reshape(NW, 32, 128), sum2.reshape(NW, 32, 128))
    return out[0, 0]


def kernel(pred, target, train_mask):
    cnt2, sum2 = _sc_hist(
        pred.reshape(-1), target.reshape(-1), train_mask.reshape(-1))
    return _tc_select(cnt2, sum2)


# speedup vs baseline: 1.2639x; 1.0305x over previous
import functools
import jax
import jax.numpy as jnp
from jax import lax
from jax.experimental import pallas as pl
from jax.experimental.pallas import tpu as pltpu
from jax.experimental.pallas import tpu_sc as plsc

NW = 32
L = 16
N = 4 * 512 * 512
PER_W = N // NW
CH = 16384
NCHUNK = PER_W // CH
NCOPY = 8
NBIN = 4096
NEG_OFF = -103 * 64
POS_OFF = NEG_OFF + 2048

_mesh = plsc.VectorSubcoreMesh(core_axis_name="c", subcore_axis_name="s")


def _sc_body(pred_hbm, t_hbm, m_hbm, cnt_out, sum_out, pred_c, t_c, m_c,
             *hists):
    wid = lax.axis_index("s") * 2 + lax.axis_index("c")
    base = wid * PER_W
    cs = tuple(zip(hists[:NCOPY], hists[NCOPY:]))

    zeros = jnp.zeros((L,), jnp.float32)

    def zero_hist(i, _):
        o = i * L
        for cv, sv in cs:
            cv[pl.ds(o, L)] = zeros
            sv[pl.ds(o, L)] = zeros
        return 0

    lax.fori_loop(0, NBIN // L, zero_hist, 0, unroll=4)

    ones = jnp.ones((L,), jnp.float32)

    def chunk(ch, _):
        off = base + ch * CH
        pltpu.sync_copy(pred_hbm.at[pl.ds(off, CH)], pred_c)
        pltpu.sync_copy(t_hbm.at[pl.ds(off, CH)], t_c)
        pltpu.sync_copy(m_hbm.at[pl.ds(off, CH)], m_c)

        def vec(i, _):
            loads = []
            for j in range(NCOPY):
                o = (i * NCOPY + j) * L
                loads.append((pred_c[pl.ds(o, L)], t_c[pl.ds(o, L)],
                              m_c[pl.ds(o, L)]))
            work = []
            for p, t, m in loads:
                tpos = t > 0
                q = jnp.maximum(jnp.where(tpos, p, 1.0 - p), 1e-7)
                key = (plsc.bitcast(q, jnp.int32) >> 17) + jnp.where(
                    tpos, POS_OFF, NEG_OFF)
                work.append((key, q, m > 0))
            for (cv, sv), (key, q, msk) in zip(cs, work):
                plsc.addupdate_scatter(cv, [key], ones, mask=msk)
                plsc.addupdate_scatter(sv, [key], q, mask=msk)
            return 0

        lax.fori_loop(0, CH // L // NCOPY, vec, 0)
        return 0

    lax.fori_loop(0, NCHUNK, chunk, 0)

    def merge(i, _):
        o = i * L
        for group in (hists[:NCOPY], hists[NCOPY:]):
            a = (group[0][pl.ds(o, L)] + group[1][pl.ds(o, L)]) + (
                group[2][pl.ds(o, L)] + group[3][pl.ds(o, L)])
            b = (group[4][pl.ds(o, L)] + group[5][pl.ds(o, L)]) + (
                group[6][pl.ds(o, L)] + group[7][pl.ds(o, L)])
            group[0][pl.ds(o, L)] = a + b
        return 0

    lax.fori_loop(0, NBIN // L, merge, 0, unroll=4)
    pltpu.sync_copy(hists[0], cnt_out.at[wid])
    pltpu.sync_copy(hists[NCOPY], sum_out.at[wid])


_sc_hist = functools.partial(
    pl.kernel, mesh=_mesh,
    out_type=(
        jax.ShapeDtypeStruct((NW, NBIN), jnp.float32),
        jax.ShapeDtypeStruct((NW, NBIN), jnp.float32),
    ),
    scratch_types=[
        pltpu.VMEM((CH,), jnp.float32),
        pltpu.VMEM((CH,), jnp.int32),
        pltpu.VMEM((CH,), jnp.int32),
    ] + [pltpu.VMEM((NBIN,), jnp.float32)] * 16,
    compiler_params=pltpu.CompilerParams(needs_layout_passes=False),
)(_sc_body)


def _tc_select_body(cnt_ref, sum_ref, out_ref):
    A = jnp.sum(cnt_ref[...], axis=0)
    Q = jnp.sum(sum_ref[...], axis=0)
    C2 = A[0:16, :]
    Q2 = Q[0:16, :]
    posC = A[16:32, :]
    posQ = Q[16:32, :]
    npos = jnp.sum(posC)
    qm_pos = jnp.where(posC > 0.0, posQ / jnp.maximum(posC, 1.0), 0.5)
    lpos = jnp.sum(posC * -jnp.log(qm_pos))
    qm_neg = jnp.where(C2 > 0.0, Q2 / jnp.maximum(C2, 1.0), 0.5)
    S2 = C2 * -jnp.log(qm_neg)
    hp = jax.lax.Precision.HIGHEST
    M1 = (lax.broadcasted_iota(jnp.int32, (128, 128), 0)
          <= lax.broadcasted_iota(jnp.int32, (128, 128), 1)).astype(jnp.float32)
    PrefC = jnp.dot(C2, M1, preferred_element_type=jnp.float32, precision=hp)
    PrefS = jnp.dot(S2, M1, preferred_element_type=jnp.float32, precision=hp)
    Arr = (lax.broadcasted_iota(jnp.int32, (16, 16), 1)
           < lax.broadcasted_iota(jnp.int32, (16, 16), 0)).astype(jnp.float32)
    RowC = jnp.dot(Arr, PrefC[:, 127:128], preferred_element_type=jnp.float32,
                   precision=hp)
    RowS = jnp.dot(Arr, PrefS[:, 127:128], preferred_element_type=jnp.float32,
                   precision=hp)
    C_geq = RowC + PrefC
    C_above = C_geq - C2
    S_above = RowS + PrefS - S2
    nneg = jnp.sum(C2)
    k = jnp.where(npos > 0.0, jnp.minimum(nneg, 3.0 * npos), 100.0)
    k_eff = jnp.minimum(k, nneg)
    sel = jnp.logical_and(C_above < k_eff, C_geq >= k_eff)
    self32 = jnp.where(sel, 1.0, 0.0) * jnp.where(k_eff > 0.0, 1.0, 0.0)
    cnt_sel = jnp.sum(self32 * C2)
    sum_sel = jnp.sum(self32 * S2)
    C_a = jnp.sum(self32 * C_above)
    S_a = jnp.sum(self32 * S_above)
    mean_sel = sum_sel / jnp.maximum(cnt_sel, 1.0)
    loss_neg = jnp.where(k_eff > 0.0, S_a + (k_eff - C_a) * mean_sel, 0.0)
    loss_neg = loss_neg + jnp.where(k > nneg, (k - nneg) * -1e30, 0.0)
    out_ref[0, 0] = (lpos + loss_neg) / (npos + k)


def _tc_select(cnt2, sum2):
    out = pl.pallas_call(
        _tc_select_body,
        out_specs=pl.BlockSpec(memory_space=pltpu.SMEM),
        out_shape=jax.ShapeDtypeStruct((1, 1), jnp.float32),
    )(cnt2.reshape(NW, 32, 128), sum2.reshape(NW, 32, 128))
    return out[0, 0]


def kernel(pred, target, train_mask):
    cnt2, sum2 = _sc_hist(
        pred.reshape(-1), target.reshape(-1), train_mask.reshape(-1))
    return _tc_select(cnt2, sum2)


# speedup vs baseline: 1.3904x; 1.1000x over previous
import functools
import jax
import jax.numpy as jnp
from jax import lax
from jax.experimental import pallas as pl
from jax.experimental.pallas import tpu as pltpu
from jax.experimental.pallas import tpu_sc as plsc

NW = 32
L = 16
N = 4 * 512 * 512
PER_W = N // NW
CH = 8192
NCHUNK = PER_W // CH
NCOPY = 8
NBIN = 4096
NEG_OFF = -103 * 64
POS_OFF = NEG_OFF + 2048

_mesh = plsc.VectorSubcoreMesh(core_axis_name="c", subcore_axis_name="s")


def _sc_body(pred_hbm, t_hbm, m_hbm, cnt_out, sum_out, pred_c0, t_c0, m_c0,
             pred_c1, t_c1, m_c1, sem0, sem1, *hists):
    wid = lax.axis_index("s") * 2 + lax.axis_index("c")
    base = wid * PER_W
    cs = tuple(zip(hists[:NCOPY], hists[NCOPY:]))

    zeros = jnp.zeros((L,), jnp.float32)

    def zero_hist(i, _):
        o = i * L
        for cv, sv in cs:
            cv[pl.ds(o, L)] = zeros
            sv[pl.ds(o, L)] = zeros
        return 0

    lax.fori_loop(0, NBIN // L, zero_hist, 0, unroll=4)

    ones = jnp.ones((L,), jnp.float32)
    bufs = ((pred_c0, t_c0, m_c0, sem0), (pred_c1, t_c1, m_c1, sem1))

    def start(ch, bset):
        off = base + ch * CH
        return (pltpu.async_copy(pred_hbm.at[pl.ds(off, CH)], bset[0], bset[3]),
                pltpu.async_copy(t_hbm.at[pl.ds(off, CH)], bset[1], bset[3]),
                pltpu.async_copy(m_hbm.at[pl.ds(off, CH)], bset[2], bset[3]))

    def compute(bset):
        pred_c, t_c, m_c = bset[0], bset[1], bset[2]

        def vec(i, _):
            loads = []
            for j in range(NCOPY):
                o = (i * NCOPY + j) * L
                loads.append((pred_c[pl.ds(o, L)], t_c[pl.ds(o, L)],
                              m_c[pl.ds(o, L)]))
            work = []
            for p, t, m in loads:
                tpos = t > 0
                q = jnp.maximum(jnp.where(tpos, p, 1.0 - p), 1e-7)
                key = (plsc.bitcast(q, jnp.int32) >> 17) + jnp.where(
                    tpos, POS_OFF, NEG_OFF)
                work.append((key, q, m > 0))
            for (cv, sv), (key, q, msk) in zip(cs, work):
                plsc.addupdate_scatter(cv, [key], ones, mask=msk)
                plsc.addupdate_scatter(sv, [key], q, mask=msk)
            return 0

        lax.fori_loop(0, CH // L // NCOPY, vec, 0)

    handles = start(0, bufs[0])
    for ch in range(NCHUNK):
        nxt = start(ch + 1, bufs[(ch + 1) % 2]) if ch + 1 < NCHUNK else None
        for h in handles:
            h.wait()
        compute(bufs[ch % 2])
        handles = nxt

    def merge(i, _):
        o = i * L
        for group in (hists[:NCOPY], hists[NCOPY:]):
            a = (group[0][pl.ds(o, L)] + group[1][pl.ds(o, L)]) + (
                group[2][pl.ds(o, L)] + group[3][pl.ds(o, L)])
            b = (group[4][pl.ds(o, L)] + group[5][pl.ds(o, L)]) + (
                group[6][pl.ds(o, L)] + group[7][pl.ds(o, L)])
            group[0][pl.ds(o, L)] = a + b
        return 0

    lax.fori_loop(0, NBIN // L, merge, 0, unroll=4)
    pltpu.sync_copy(hists[0], cnt_out.at[wid])
    pltpu.sync_copy(hists[NCOPY], sum_out.at[wid])


_sc_hist = functools.partial(
    pl.kernel, mesh=_mesh,
    out_type=(
        jax.ShapeDtypeStruct((NW, NBIN), jnp.float32),
        jax.ShapeDtypeStruct((NW, NBIN), jnp.float32),
    ),
    scratch_types=[
        pltpu.VMEM((CH,), jnp.float32),
        pltpu.VMEM((CH,), jnp.int32),
        pltpu.VMEM((CH,), jnp.int32),
        pltpu.VMEM((CH,), jnp.float32),
        pltpu.VMEM((CH,), jnp.int32),
        pltpu.VMEM((CH,), jnp.int32),
        pltpu.SemaphoreType.DMA,
        pltpu.SemaphoreType.DMA,
    ] + [pltpu.VMEM((NBIN,), jnp.float32)] * 16,
    compiler_params=pltpu.CompilerParams(needs_layout_passes=False),
)(_sc_body)


def _tc_select_body(cnt_ref, sum_ref, out_ref):
    A = jnp.sum(cnt_ref[...], axis=0)
    Q = jnp.sum(sum_ref[...], axis=0)
    C2 = A[0:16, :]
    Q2 = Q[0:16, :]
    posC = A[16:32, :]
    posQ = Q[16:32, :]
    npos = jnp.sum(posC)
    qm_pos = jnp.where(posC > 0.0, posQ / jnp.maximum(posC, 1.0), 0.5)
    lpos = jnp.sum(posC * -jnp.log(qm_pos))
    qm_neg = jnp.where(C2 > 0.0, Q2 / jnp.maximum(C2, 1.0), 0.5)
    S2 = C2 * -jnp.log(qm_neg)
    hp = jax.lax.Precision.HIGHEST
    M1 = (lax.broadcasted_iota(jnp.int32, (128, 128), 0)
          <= lax.broadcasted_iota(jnp.int32, (128, 128), 1)).astype(jnp.float32)
    PrefC = jnp.dot(C2, M1, preferred_element_type=jnp.float32, precision=hp)
    PrefS = jnp.dot(S2, M1, preferred_element_type=jnp.float32, precision=hp)
    Arr = (lax.broadcasted_iota(jnp.int32, (16, 16), 1)
           < lax.broadcasted_iota(jnp.int32, (16, 16), 0)).astype(jnp.float32)
    RowC = jnp.dot(Arr, PrefC[:, 127:128], preferred_element_type=jnp.float32,
                   precision=hp)
    RowS = jnp.dot(Arr, PrefS[:, 127:128], preferred_element_type=jnp.float32,
                   precision=hp)
    C_geq = RowC + PrefC
    C_above = C_geq - C2
    S_above = RowS + PrefS - S2
    nneg = jnp.sum(C2)
    k = jnp.where(npos > 0.0, jnp.minimum(nneg, 3.0 * npos), 100.0)
    k_eff = jnp.minimum(k, nneg)
    sel = jnp.logical_and(C_above < k_eff, C_geq >= k_eff)
    self32 = jnp.where(sel, 1.0, 0.0) * jnp.where(k_eff > 0.0, 1.0, 0.0)
    cnt_sel = jnp.sum(self32 * C2)
    sum_sel = jnp.sum(self32 * S2)
    C_a = jnp.sum(self32 * C_above)
    S_a = jnp.sum(self32 * S_above)
    mean_sel = sum_sel / jnp.maximum(cnt_sel, 1.0)
    loss_neg = jnp.where(k_eff > 0.0, S_a + (k_eff - C_a) * mean_sel, 0.0)
    loss_neg = loss_neg + jnp.where(k > nneg, (k - nneg) * -1e30, 0.0)
    out_ref[0, 0] = (lpos + loss_neg) / (npos + k)


def _tc_select(cnt2, sum2):
    out = pl.pallas_call(
        _tc_select_body,
        out_specs=pl.BlockSpec(memory_space=pltpu.SMEM),
        out_shape=jax.ShapeDtypeStruct((1, 1), jnp.float32),
    )(cnt2.reshape(NW, 32, 128), sum2.reshape(NW, 32, 128))
    return out[0, 0]


def kernel(pred, target, train_mask):
    cnt2, sum2 = _sc_hist(
        pred.reshape(-1), target.reshape(-1), train_mask.reshape(-1))
    return _tc_select(cnt2, sum2)


# speedup vs baseline: 1.3934x; 1.0021x over previous
import functools
import jax
import jax.numpy as jnp
from jax import lax
from jax.experimental import pallas as pl
from jax.experimental.pallas import tpu as pltpu
from jax.experimental.pallas import tpu_sc as plsc

NW = 32
L = 16
N = 4 * 512 * 512
PER_W = N // NW
CH = 8192
NCHUNK = PER_W // CH
NCOPY = 8
NBIN = 4096
NEG_OFF = -103 * 64
POS_OFF = NEG_OFF + 2048

_mesh = plsc.VectorSubcoreMesh(core_axis_name="c", subcore_axis_name="s")


def _sc_body(pred_hbm, t_hbm, m_hbm, cnt_out, sum_out, pred_c0, t_c0, m_c0,
             pred_c1, t_c1, m_c1, sem0, sem1, *hists):
    wid = lax.axis_index("s") * 2 + lax.axis_index("c")
    base = wid * PER_W
    cs = tuple(zip(hists[:NCOPY], hists[NCOPY:]))

    zeros = jnp.zeros((L,), jnp.float32)

    def zero_hist(i, _):
        o = i * L
        for cv, sv in cs:
            cv[pl.ds(o, L)] = zeros
            sv[pl.ds(o, L)] = zeros
        return 0

    lax.fori_loop(0, NBIN // L, zero_hist, 0, unroll=4)

    ones = jnp.ones((L,), jnp.float32)
    bufs = ((pred_c0, t_c0, m_c0, sem0), (pred_c1, t_c1, m_c1, sem1))

    def start(ch, bset):
        off = base + ch * CH
        return (pltpu.async_copy(pred_hbm.at[pl.ds(off, CH)], bset[0], bset[3]),
                pltpu.async_copy(t_hbm.at[pl.ds(off, CH)], bset[1], bset[3]),
                pltpu.async_copy(m_hbm.at[pl.ds(off, CH)], bset[2], bset[3]))

    def compute(bset):
        pred_c, t_c, m_c = bset[0], bset[1], bset[2]

        def vec(i, _):
            loads = []
            for j in range(NCOPY):
                o = (i * NCOPY + j) * L
                loads.append((pred_c[pl.ds(o, L)], t_c[pl.ds(o, L)],
                              m_c[pl.ds(o, L)]))
            work = []
            for p, t, m in loads:
                tpos = t > 0
                q = jnp.maximum(jnp.where(tpos, p, 1.0 - p),
                                jnp.where(tpos, 1e-7, 1.1920928955078125e-07))
                key = (plsc.bitcast(q, jnp.int32) >> 17) + jnp.where(
                    tpos, POS_OFF, NEG_OFF)
                work.append((key, q, m > 0))
            for (cv, sv), (key, q, msk) in zip(cs, work):
                plsc.addupdate_scatter(cv, [key], ones, mask=msk)
                plsc.addupdate_scatter(sv, [key], q, mask=msk)
            return 0

        lax.fori_loop(0, CH // L // NCOPY, vec, 0)

    handles = start(0, bufs[0])
    for ch in range(NCHUNK):
        nxt = start(ch + 1, bufs[(ch + 1) % 2]) if ch + 1 < NCHUNK else None
        for h in handles:
            h.wait()
        compute(bufs[ch % 2])
        handles = nxt

    def merge(i, _):
        o = i * L
        for group in (hists[:NCOPY], hists[NCOPY:]):
            a = (group[0][pl.ds(o, L)] + group[1][pl.ds(o, L)]) + (
                group[2][pl.ds(o, L)] + group[3][pl.ds(o, L)])
            b = (group[4][pl.ds(o, L)] + group[5][pl.ds(o, L)]) + (
                group[6][pl.ds(o, L)] + group[7][pl.ds(o, L)])
            group[0][pl.ds(o, L)] = a + b
        return 0

    lax.fori_loop(0, NBIN // L, merge, 0, unroll=4)
    pltpu.sync_copy(hists[0], cnt_out.at[wid])
    pltpu.sync_copy(hists[NCOPY], sum_out.at[wid])


_sc_hist = functools.partial(
    pl.kernel, mesh=_mesh,
    out_type=(
        jax.ShapeDtypeStruct((NW, NBIN), jnp.float32),
        jax.ShapeDtypeStruct((NW, NBIN), jnp.float32),
    ),
    scratch_types=[
        pltpu.VMEM((CH,), jnp.float32),
        pltpu.VMEM((CH,), jnp.int32),
        pltpu.VMEM((CH,), jnp.int32),
        pltpu.VMEM((CH,), jnp.float32),
        pltpu.VMEM((CH,), jnp.int32),
        pltpu.VMEM((CH,), jnp.int32),
        pltpu.SemaphoreType.DMA,
        pltpu.SemaphoreType.DMA,
    ] + [pltpu.VMEM((NBIN,), jnp.float32)] * 16,
    compiler_params=pltpu.CompilerParams(needs_layout_passes=False),
)(_sc_body)


def _tc_select_body(cnt_ref, sum_ref, out_ref):
    A = jnp.sum(cnt_ref[...], axis=0)
    Q = jnp.sum(sum_ref[...], axis=0)
    C2 = A[0:16, :]
    Q2 = Q[0:16, :]
    posC = A[16:32, :]
    posQ = Q[16:32, :]
    npos = jnp.sum(posC)
    qm_pos = jnp.where(posC > 0.0, posQ / jnp.maximum(posC, 1.0), 0.5)
    lpos = jnp.sum(posC * -jnp.log(qm_pos))
    qm_neg = jnp.where(C2 > 0.0, Q2 / jnp.maximum(C2, 1.0), 0.5)
    S2 = C2 * -jnp.log(qm_neg)
    hp = jax.lax.Precision.HIGHEST
    M1 = (lax.broadcasted_iota(jnp.int32, (128, 128), 0)
          <= lax.broadcasted_iota(jnp.int32, (128, 128), 1)).astype(jnp.float32)
    PrefC = jnp.dot(C2, M1, preferred_element_type=jnp.float32, precision=hp)
    PrefS = jnp.dot(S2, M1, preferred_element_type=jnp.float32, precision=hp)
    Arr = (lax.broadcasted_iota(jnp.int32, (16, 16), 1)
           < lax.broadcasted_iota(jnp.int32, (16, 16), 0)).astype(jnp.float32)
    RowC = jnp.dot(Arr, PrefC[:, 127:128], preferred_element_type=jnp.float32,
                   precision=hp)
    RowS = jnp.dot(Arr, PrefS[:, 127:128], preferred_element_type=jnp.float32,
                   precision=hp)
    C_geq = RowC + PrefC
    C_above = C_geq - C2
    S_above = RowS + PrefS - S2
    nneg = jnp.sum(C2)
    k = jnp.where(npos > 0.0, jnp.minimum(nneg, 3.0 * npos), 100.0)
    k_eff = jnp.minimum(k, nneg)
    sel = jnp.logical_and(C_above < k_eff, C_geq >= k_eff)
    self32 = jnp.where(sel, 1.0, 0.0) * jnp.where(k_eff > 0.0, 1.0, 0.0)
    cnt_sel = jnp.sum(self32 * C2)
    sum_sel = jnp.sum(self32 * S2)
    C_a = jnp.sum(self32 * C_above)
    S_a = jnp.sum(self32 * S_above)
    mean_sel = sum_sel / jnp.maximum(cnt_sel, 1.0)
    loss_neg = jnp.where(k_eff > 0.0, S_a + (k_eff - C_a) * mean_sel, 0.0)
    loss_neg = loss_neg + jnp.where(k > nneg, (k - nneg) * -1e30, 0.0)
    out_ref[0, 0] = (lpos + loss_neg) / (npos + k)


def _tc_select(cnt2, sum2):
    out = pl.pallas_call(
        _tc_select_body,
        out_specs=pl.BlockSpec(memory_space=pltpu.SMEM),
        out_shape=jax.ShapeDtypeStruct((1, 1), jnp.float32),
    )(cnt2.reshape(NW, 32, 128), sum2.reshape(NW, 32, 128))
    return out[0, 0]


def kernel(pred, target, train_mask):
    cnt2, sum2 = _sc_hist(
        pred.reshape(-1), target.reshape(-1), train_mask.reshape(-1))
    return _tc_select(cnt2, sum2)
